# baseline (speedup 1.0000x reference)
import functools

import jax
import jax.numpy as jnp
from jax import lax
from jax.experimental import pallas as pl
from jax.experimental.pallas import tpu as pltpu
from jax.experimental.pallas import tpu_sc as plsc

_NUM_CCSK = 8
_SEQ = 128
_F = _NUM_CCSK * _SEQ
_NC = 2
_NS = 16
_NW = _NC * _NS
_CH = 16


def _permute_chunk(in_v, out_v):
    lane = lax.iota(jnp.int32, 16)

    def j_body(j, _):
        for k in range(_NUM_CCSK):
            src = jnp.broadcast_to(k * _SEQ + j, (16,))
            dst = jnp.broadcast_to(j * _NUM_CCSK + k, (16,))
            v = plsc.load_gather(in_v, [lane, src])
            plsc.store_scatter(out_v, [lane, dst], v)
        return 0

    lax.fori_loop(0, _SEQ, j_body, 0)


def _sc_body(x_hbm, out_hbm, in_a, in_b, out_a, out_b, si_a, si_b, so_a, so_b):
    wid = lax.axis_index("s") * _NC + lax.axis_index("c")
    rows_per_w = x_hbm.shape[0] // _NW
    base = wid * rows_per_w
    n = rows_per_w // _CH

    ins = [in_a, in_b]
    outs = [out_a, out_b]
    sem_in = [si_a, si_b]
    sem_out = [so_a, so_b]

    def start_in(i):
        r0 = base + i * _CH
        return pltpu.async_copy(
            x_hbm.at[pl.ds(r0, _CH)], ins[i % 2].at[:, pl.ds(0, _F)],
            sem_in[i % 2]
        )

    def start_out(i):
        r0 = base + i * _CH
        return pltpu.async_copy(
            outs[i % 2].at[:, pl.ds(0, _F)], out_hbm.at[pl.ds(r0, _CH)],
            sem_out[i % 2]
        )

    in_descs = {0: start_in(0), 1: start_in(1)}
    out_descs = {}
    for i in range(n):
        in_descs.pop(i).wait()
        if i >= 2:
            out_descs.pop(i - 2).wait()
        _permute_chunk(ins[i % 2], outs[i % 2])
        out_descs[i] = start_out(i)
        if i + 2 < n:
            in_descs[i + 2] = start_in(i + 2)
    out_descs.pop(n - 2).wait()
    out_descs.pop(n - 1).wait()


def kernel(inputs):
    b, t, f = inputs.shape
    rows = b * t
    x = inputs.reshape(rows, f)
    pitch = f + 1
    mesh = plsc.VectorSubcoreMesh(core_axis_name="c", subcore_axis_name="s")
    k = functools.partial(
        pl.kernel,
        out_type=jax.ShapeDtypeStruct((rows, f), jnp.float32),
        mesh=mesh,
        scratch_types=[
            pltpu.VMEM((_CH, pitch), jnp.float32),
            pltpu.VMEM((_CH, pitch), jnp.float32),
            pltpu.VMEM((_CH, pitch), jnp.float32),
            pltpu.VMEM((_CH, pitch), jnp.float32),
            pltpu.SemaphoreType.DMA,
            pltpu.SemaphoreType.DMA,
            pltpu.SemaphoreType.DMA,
            pltpu.SemaphoreType.DMA,
        ],
        compiler_params=pltpu.CompilerParams(needs_layout_passes=False),
    )(_sc_body)
    out = k(x)
    return out.reshape(b, t, f)


# speedup vs baseline: 2.3866x; 2.3866x over previous
import functools

import jax
import jax.numpy as jnp
from jax import lax
from jax.experimental import pallas as pl
from jax.experimental.pallas import tpu as pltpu
from jax.experimental.pallas import tpu_sc as plsc

_NUM_CCSK = 8
_SEQ = 128
_F = _NUM_CCSK * _SEQ
_NC = 2
_NS = 16
_NW = _NC * _NS
_CH = 16


_PITCH = 1032


def _permute_chunk(in_v, out_v):
    lane = lax.iota(jnp.int32, 16)
    lanebase = (lane << 10) + (lane << 3)

    def j_body(j, _):
        vs = [
            plsc.load_gather(in_v, [lanebase + (k * _SEQ + j)])
            for k in range(_NUM_CCSK)
        ]
        for k in range(_NUM_CCSK):
            plsc.store_scatter(out_v, [lanebase + (j * _NUM_CCSK + k)], vs[k])
        return 0

    lax.fori_loop(0, _SEQ, j_body, 0, unroll=2)


def _sc_body(x_hbm, out_hbm, in_a, in_b, out_a, out_b, si_a, si_b, so_a, so_b):
    wid = lax.axis_index("s") * _NC + lax.axis_index("c")
    rows_per_w = x_hbm.shape[0] // _F // _NW
    base = wid * rows_per_w
    n = rows_per_w // _CH

    ins = [in_a, in_b]
    outs = [out_a, out_b]
    sem_in = [si_a, si_b]
    sem_out = [so_a, so_b]

    def start_in(i, b):
        r0 = base + i * _CH
        for rr in range(_CH):
            pltpu.async_copy(
                x_hbm.at[pl.ds((r0 + rr) * _F, _F)],
                ins[b].at[pl.ds(rr * _PITCH, _F)],
                sem_in[b],
            )

    def start_out(i, b):
        r0 = base + i * _CH
        for rr in range(_CH):
            pltpu.async_copy(
                outs[b].at[pl.ds(rr * _PITCH, _F)],
                out_hbm.at[pl.ds((r0 + rr) * _F, _F)],
                sem_out[b],
            )

    def wait_in(b):
        pltpu.make_async_copy(
            x_hbm.at[pl.ds(0, _CH * _F)],
            ins[b].at[pl.ds(0, _CH * _F)],
            sem_in[b],
        ).wait()

    def wait_out(b):
        pltpu.make_async_copy(
            outs[b].at[pl.ds(0, _CH * _F)],
            out_hbm.at[pl.ds(0, _CH * _F)],
            sem_out[b],
        ).wait()

    start_in(0, 0)
    start_in(1, 1)
    wait_in(0)
    _permute_chunk(ins[0], outs[0])
    start_out(0, 0)
    start_in(2, 0)
    wait_in(1)
    _permute_chunk(ins[1], outs[1])
    start_out(1, 1)
    start_in(3, 1)

    def pair_body(g, _):
        i0 = 2 * g
        wait_in(0)
        wait_out(0)
        _permute_chunk(ins[0], outs[0])
        start_out(i0, 0)
        start_in(i0 + 2, 0)
        wait_in(1)
        wait_out(1)
        _permute_chunk(ins[1], outs[1])
        start_out(i0 + 1, 1)
        start_in(i0 + 3, 1)
        return 0

    lax.fori_loop(1, n // 2 - 1, pair_body, 0)

    wait_in(0)
    wait_out(0)
    _permute_chunk(ins[0], outs[0])
    start_out(n - 2, 0)
    wait_in(1)
    wait_out(1)
    _permute_chunk(ins[1], outs[1])
    start_out(n - 1, 1)
    wait_out(0)
    wait_out(1)


def kernel(inputs):
    b, t, f = inputs.shape
    rows = b * t
    x = inputs.reshape(rows * f)
    mesh = plsc.VectorSubcoreMesh(core_axis_name="c", subcore_axis_name="s")
    k = functools.partial(
        pl.kernel,
        out_type=jax.ShapeDtypeStruct((rows * f,), jnp.float32),
        mesh=mesh,
        scratch_types=[
            pltpu.VMEM((_CH * _PITCH,), jnp.float32),
            pltpu.VMEM((_CH * _PITCH,), jnp.float32),
            pltpu.VMEM((_CH * _PITCH,), jnp.float32),
            pltpu.VMEM((_CH * _PITCH,), jnp.float32),
            pltpu.SemaphoreType.DMA,
            pltpu.SemaphoreType.DMA,
            pltpu.SemaphoreType.DMA,
            pltpu.SemaphoreType.DMA,
        ],
        compiler_params=pltpu.CompilerParams(needs_layout_passes=False),
    )(_sc_body)
    out = k(x)
    return out.reshape(b, t, f)


# speedup vs baseline: 2.7142x; 1.1373x over previous
import functools

import jax
import jax.numpy as jnp
from jax import lax
from jax.experimental import pallas as pl
from jax.experimental.pallas import tpu as pltpu
from jax.experimental.pallas import tpu_sc as plsc

_NUM_CCSK = 8
_SEQ = 128
_F = _NUM_CCSK * _SEQ
_NC = 2
_NS = 16
_NW = _NC * _NS
_CH = 16


def _permute_chunk(in_v, out_v):
    pattern = lax.iota(jnp.int32, 16) << 3

    def t_body(t, _):
        off = (t >> 3) << 10
        m = t & 7
        vs = [
            in_v[pl.ds(off + m * 16 + k * _SEQ, 16)]
            for k in range(_NUM_CCSK)
        ]
        for k in range(_NUM_CCSK):
            plsc.store_scatter(
                out_v, [pattern + (off + m * _SEQ + k)], vs[k]
            )
        return 0

    lax.fori_loop(0, _CH * _NUM_CCSK, t_body, 0, unroll=2)


def _sc_body(x_hbm, out_hbm, in_a, in_b, out_a, out_b, si_a, si_b, so_a, so_b):
    wid = lax.axis_index("s") * _NC + lax.axis_index("c")
    rows_per_w = x_hbm.shape[0] // _F // _NW
    base = wid * rows_per_w
    n = rows_per_w // _CH

    ins = [in_a, in_b]
    outs = [out_a, out_b]
    sem_in = [si_a, si_b]
    sem_out = [so_a, so_b]

    def start_in(i, b):
        r0 = base + i * _CH
        pltpu.async_copy(
            x_hbm.at[pl.ds(r0 * _F, _CH * _F)], ins[b], sem_in[b]
        )

    def start_out(i, b):
        r0 = base + i * _CH
        pltpu.async_copy(
            outs[b], out_hbm.at[pl.ds(r0 * _F, _CH * _F)], sem_out[b]
        )

    def wait_in(b):
        pltpu.make_async_copy(
            x_hbm.at[pl.ds(0, _CH * _F)], ins[b], sem_in[b]
        ).wait()

    def wait_out(b):
        pltpu.make_async_copy(
            outs[b], out_hbm.at[pl.ds(0, _CH * _F)], sem_out[b]
        ).wait()

    start_in(0, 0)
    start_in(1, 1)
    wait_in(0)
    _permute_chunk(ins[0], outs[0])
    start_out(0, 0)
    start_in(2, 0)
    wait_in(1)
    _permute_chunk(ins[1], outs[1])
    start_out(1, 1)
    start_in(3, 1)

    def pair_body(g, _):
        i0 = 2 * g
        wait_in(0)
        wait_out(0)
        _permute_chunk(ins[0], outs[0])
        start_out(i0, 0)
        start_in(i0 + 2, 0)
        wait_in(1)
        wait_out(1)
        _permute_chunk(ins[1], outs[1])
        start_out(i0 + 1, 1)
        start_in(i0 + 3, 1)
        return 0

    lax.fori_loop(1, n // 2 - 1, pair_body, 0)

    wait_in(0)
    wait_out(0)
    _permute_chunk(ins[0], outs[0])
    start_out(n - 2, 0)
    wait_in(1)
    wait_out(1)
    _permute_chunk(ins[1], outs[1])
    start_out(n - 1, 1)
    wait_out(0)
    wait_out(1)


def kernel(inputs):
    b, t, f = inputs.shape
    rows = b * t
    x = inputs.reshape(rows * f)
    mesh = plsc.VectorSubcoreMesh(core_axis_name="c", subcore_axis_name="s")
    k = functools.partial(
        pl.kernel,
        out_type=jax.ShapeDtypeStruct((rows * f,), jnp.float32),
        mesh=mesh,
        scratch_types=[
            pltpu.VMEM((_CH * _F,), jnp.float32),
            pltpu.VMEM((_CH * _F,), jnp.float32),
            pltpu.VMEM((_CH * _F,), jnp.float32),
            pltpu.VMEM((_CH * _F,), jnp.float32),
            pltpu.SemaphoreType.DMA,
            pltpu.SemaphoreType.DMA,
            pltpu.SemaphoreType.DMA,
            pltpu.SemaphoreType.DMA,
        ],
        compiler_params=pltpu.CompilerParams(needs_layout_passes=False),
    )(_sc_body)
    out = k(x)
    return out.reshape(b, t, f)


# speedup vs baseline: 11.4568x; 4.2210x over previous
import functools

import jax
import jax.numpy as jnp
from jax import lax
from jax.experimental import pallas as pl
from jax.experimental.pallas import tpu as pltpu
from jax.experimental.pallas import tpu_sc as plsc

_NUM_CCSK = 8
_SEQ = 128
_F = _NUM_CCSK * _SEQ
_NC = 2
_NS = 16
_NW = _NC * _NS
_CH = 16


def _permute_chunk(in_v, out_v):
    pattern = lax.iota(jnp.int32, 16) << 3

    def q_body(q, _):
        r = q >> 3
        jh = q & 7
        rvec = jnp.broadcast_to(r, (16,))
        vs = [
            in_v[r, pl.ds(k * _SEQ + 16 * jh, 16)]
            for k in range(_NUM_CCSK)
        ]
        for k in range(_NUM_CCSK):
            plsc.store_scatter(
                out_v, [rvec, pattern + (jh * _SEQ + k)], vs[k]
            )
        return 0

    lax.fori_loop(0, _CH * _NUM_CCSK, q_body, 0, unroll=2)


def _sc_body(x_hbm, out_hbm, in_a, in_b, out_a, out_b, si_a, si_b, so_a, so_b):
    wid = lax.axis_index("s") * _NC + lax.axis_index("c")
    rows = x_hbm.shape[0]
    n = rows // _CH // _NW
    r_base = wid * n * _CH

    ins = [in_a, in_b]
    outs = [out_a, out_b]
    sem_in = [si_a, si_b]
    sem_out = [so_a, so_b]

    def start_in(i, b):
        r0 = r_base + i * _CH
        pltpu.async_copy(x_hbm.at[pl.ds(r0, _CH)], ins[b], sem_in[b])

    def start_out(i, b):
        r0 = r_base + i * _CH
        pltpu.async_copy(outs[b], out_hbm.at[pl.ds(r0, _CH)], sem_out[b])

    def wait_in(b):
        pltpu.make_async_copy(
            x_hbm.at[pl.ds(0, _CH)], ins[b], sem_in[b]
        ).wait()

    def wait_out(b):
        pltpu.make_async_copy(
            outs[b], out_hbm.at[pl.ds(0, _CH)], sem_out[b]
        ).wait()

    start_in(0, 0)
    start_in(1, 1)
    wait_in(0)
    _permute_chunk(ins[0], outs[0])
    start_out(0, 0)
    start_in(2, 0)
    wait_in(1)
    _permute_chunk(ins[1], outs[1])
    start_out(1, 1)
    start_in(3, 1)

    def pair_body(g, _):
        i0 = 2 * g
        wait_in(0)
        wait_out(0)
        _permute_chunk(ins[0], outs[0])
        start_out(i0, 0)
        start_in(i0 + 2, 0)
        wait_in(1)
        wait_out(1)
        _permute_chunk(ins[1], outs[1])
        start_out(i0 + 1, 1)
        start_in(i0 + 3, 1)
        return 0

    lax.fori_loop(1, n // 2 - 1, pair_body, 0)

    wait_in(0)
    wait_out(0)
    _permute_chunk(ins[0], outs[0])
    start_out(n - 2, 0)
    wait_in(1)
    wait_out(1)
    _permute_chunk(ins[1], outs[1])
    start_out(n - 1, 1)
    wait_out(0)
    wait_out(1)


def kernel(inputs):
    b, t, f = inputs.shape
    xt = jnp.transpose(inputs, (1, 0, 2)).reshape(t * b, f)
    mesh = plsc.VectorSubcoreMesh(core_axis_name="c", subcore_axis_name="s")
    k = functools.partial(
        pl.kernel,
        out_type=jax.ShapeDtypeStruct((t * b, f), jnp.float32),
        mesh=mesh,
        scratch_types=[
            pltpu.VMEM((_CH, f), jnp.float32),
            pltpu.VMEM((_CH, f), jnp.float32),
            pltpu.VMEM((_CH, f), jnp.float32),
            pltpu.VMEM((_CH, f), jnp.float32),
            pltpu.SemaphoreType.DMA,
            pltpu.SemaphoreType.DMA,
            pltpu.SemaphoreType.DMA,
            pltpu.SemaphoreType.DMA,
        ],
        compiler_params=pltpu.CompilerParams(
            needs_layout_passes=False, use_tc_tiling_on_sc=True
        ),
    )(_sc_body)
    return jnp.transpose(k(xt).reshape(t, b, f), (1, 0, 2))
